# baseline (speedup 1.0000x reference)
import functools
import math

import jax
import jax.numpy as jnp
from jax import lax
from jax.experimental import pallas as pl
from jax.experimental.pallas import tpu as pltpu
from jax.experimental.pallas import tpu_sc as plsc

VOCAB = 1000000
D = 64
SCALE = math.sqrt(D)

_info = plsc.get_sparse_core_info()
NC, NS, L = _info.num_cores, _info.num_subcores, _info.num_lanes
NW = NC * NS

CHUNK = 128
B_TOTAL = 4096 * 200
B_PER_W = B_TOTAL // NW
N_UNITS = B_PER_W // CHUNK
S_CHUNKS = 4096 // CHUNK
NBUF = 4
INBUF = 2 * NBUF
N_GROUPS = N_UNITS // INBUF


def _sc_kernel(idx_hbm, tbl_hbm, out_hbm, *scratch):
    ibufs = scratch[0:INBUF]
    gbufs = scratch[INBUF:INBUF + NBUF]
    sbufs = scratch[INBUF + NBUF:INBUF + 2 * NBUF]
    isems = scratch[INBUF + 2 * NBUF:2 * INBUF + 2 * NBUF]
    gsems = scratch[2 * INBUF + 2 * NBUF:2 * INBUF + 3 * NBUF]
    ssems = scratch[2 * INBUF + 3 * NBUF:2 * INBUF + 4 * NBUF]

    wid = lax.axis_index("s") * NC + lax.axis_index("c")
    iota = lax.iota(jnp.int32, 16)
    zero16 = jnp.full((16,), 0, jnp.int32)

    def unit_sc(u):
        uid = wid * N_UNITS + u
        return lax.div(uid, S_CHUNKS), lax.rem(uid, S_CHUNKS)

    def icopy_start(u, ib):
        s, c = unit_sc(u)
        src = idx_hbm.at[lax.div(s, 8), c, lax.rem(s, 8), :]
        pltpu.async_copy(src, ibufs[ib], isems[ib])

    def icopy_wait(ib):
        pltpu.make_async_copy(idx_hbm.at[0, 0, 0, :], ibufs[ib],
                              isems[ib]).wait()

    def gather_start(ib, gb):
        pltpu.async_copy(tbl_hbm.at[ibufs[ib]], gbufs[gb], gsems[gb])

    def gather_wait(gb):
        pltpu.make_async_copy(tbl_hbm.at[ibufs[0]], gbufs[gb],
                              gsems[gb]).wait()

    def out_slice(u):
        s, c = unit_sc(u)
        return out_hbm.at[s, :, c, :, :]

    def store_start(u, gb):
        pltpu.async_copy(sbufs[gb], out_slice(u), ssems[gb])

    def store_wait(u, gb):
        pltpu.make_async_copy(sbufs[gb], out_slice(u), ssems[gb]).wait()

    def transpose_scale(gb):
        g, s = gbufs[gb], sbufs[gb]
        for a in range(CHUNK // 16):
            rows = iota + (a * 16)

            def feat_body(f, _):
                vals = plsc.load_gather(g, [rows, zero16 + f]) * SCALE
                s[lax.shift_right_logical(f, 3), jnp.bitwise_and(f, 7),
                  pl.ds(a * 16, 16)] = vals
                return 0

            lax.fori_loop(0, D, feat_body, 0)

    for v in range(INBUF):
        icopy_start(v, v)
    for v in range(NBUF):
        icopy_wait(v)
        gather_start(v, v)

    def group_body(grp, _):
        for b in range(INBUF):
            u = grp * INBUF + b
            gb = b % NBUF

            gather_wait(gb)

            if b < NBUF:
                @pl.when(grp > 0)
                def _():
                    store_wait(u - NBUF, gb)
            else:
                store_wait(u - NBUF, gb)

            transpose_scale(gb)
            store_start(u, gb)

            @pl.when(grp < N_GROUPS - 1)
            def _():
                icopy_start(u + INBUF, b)

            nb = (b + NBUF) % INBUF
            if b < NBUF:
                icopy_wait(nb)
                gather_start(nb, gb)
            else:
                @pl.when(grp < N_GROUPS - 1)
                def _():
                    icopy_wait(nb)
                    gather_start(nb, gb)

        return 0

    lax.fori_loop(0, N_GROUPS, group_body, 0)
    for b in range(NBUF):
        store_wait(N_UNITS - NBUF + b, b)


@jax.jit
def kernel(x, table):
    idx4 = x.T.reshape(25, 8, 32, CHUNK).transpose(0, 2, 1, 3)

    mesh = plsc.VectorSubcoreMesh(core_axis_name="c", subcore_axis_name="s")

    out5 = pl.kernel(
        _sc_kernel,
        mesh=mesh,
        compiler_params=pltpu.CompilerParams(use_tc_tiling_on_sc=False,
                                             needs_layout_passes=False),
        out_type=jax.ShapeDtypeStruct((200, 8, S_CHUNKS, 8, CHUNK),
                                      jnp.float32),
        scratch_types=(
            [pltpu.VMEM((CHUNK,), jnp.int32) for _ in range(INBUF)]
            + [pltpu.VMEM((CHUNK, D), jnp.float32) for _ in range(NBUF)]
            + [pltpu.VMEM((8, 8, CHUNK), jnp.float32) for _ in range(NBUF)]
            + [pltpu.SemaphoreType.DMA for _ in range(INBUF + 2 * NBUF)]
        ),
    )(idx4, table)

    out = out5.transpose(0, 1, 3, 2, 4).reshape(200, D, 4096)
    return out.transpose(2, 0, 1)


# speedup vs baseline: 1.1244x; 1.1244x over previous
import functools
import math

import jax
import jax.numpy as jnp
from jax import lax
from jax.experimental import pallas as pl
from jax.experimental.pallas import tpu as pltpu
from jax.experimental.pallas import tpu_sc as plsc

VOCAB = 1000000
D = 64
SCALE = math.sqrt(D)

_info = plsc.get_sparse_core_info()
NC, NS, L = _info.num_cores, _info.num_subcores, _info.num_lanes
NW = NC * NS

CHUNK = 128
B_TOTAL = 4096 * 200
B_PER_W = B_TOTAL // NW
N_UNITS = B_PER_W // CHUNK
S_CHUNKS = 4096 // CHUNK
NBUF = 4
INBUF = 2 * NBUF
N_GROUPS = N_UNITS // INBUF


def _sc_kernel(idx_hbm, tbl_hbm, out_hbm, *scratch):
    ibufs = scratch[0:INBUF]
    gbufs = scratch[INBUF:INBUF + NBUF]
    sbufs = scratch[INBUF + NBUF:INBUF + 2 * NBUF]
    isems = scratch[INBUF + 2 * NBUF:2 * INBUF + 2 * NBUF]
    gsems = scratch[2 * INBUF + 2 * NBUF:2 * INBUF + 3 * NBUF]
    ssems = scratch[2 * INBUF + 3 * NBUF:2 * INBUF + 4 * NBUF]

    wid = lax.axis_index("s") * NC + lax.axis_index("c")
    iota = lax.iota(jnp.int32, 16)
    zero16 = jnp.full((16,), 0, jnp.int32)

    def unit_sc(u):
        uid = wid * N_UNITS + u
        return lax.div(uid, S_CHUNKS), lax.rem(uid, S_CHUNKS)

    def icopy_start(u, ib):
        s, c = unit_sc(u)
        src = idx_hbm.at[lax.div(s, 8), c, lax.rem(s, 8), :]
        pltpu.async_copy(src, ibufs[ib], isems[ib])

    def icopy_wait(ib):
        pltpu.make_async_copy(idx_hbm.at[0, 0, 0, :], ibufs[ib],
                              isems[ib]).wait()

    def gather_start(ib, gb):
        pltpu.async_copy(tbl_hbm.at[ibufs[ib]], gbufs[gb], gsems[gb])

    def gather_wait(gb):
        pltpu.make_async_copy(tbl_hbm.at[ibufs[0]], gbufs[gb],
                              gsems[gb]).wait()

    def store_start(u, gb):
        s, c = unit_sc(u)
        for tf in range(8):
            pltpu.async_copy(sbufs[gb].at[pl.ds(8 * tf, 8), :],
                             out_hbm.at[s, tf, c, :, :], ssems[gb])

    def store_wait(u, gb):
        s, c = unit_sc(u)
        for tf in range(8):
            pltpu.make_async_copy(sbufs[gb].at[pl.ds(8 * tf, 8), :],
                                  out_hbm.at[s, tf, c, :, :],
                                  ssems[gb]).wait()

    fvecs = [iota + (16 * fb) for fb in range(D // 16)]

    def transpose_scale(gb):
        g, s = gbufs[gb], sbufs[gb]

        def j_body(j, _):
            jsplat = zero16 + j
            for fb in range(D // 16):
                vals = g[j, pl.ds(16 * fb, 16)] * SCALE
                plsc.store_scatter(s, [fvecs[fb], jsplat], vals)
            return 0

        lax.fori_loop(0, CHUNK, j_body, 0, unroll=2)

    for v in range(INBUF):
        icopy_start(v, v)
    for v in range(NBUF):
        icopy_wait(v)
        gather_start(v, v)

    def group_body(grp, _):
        for b in range(INBUF):
            u = grp * INBUF + b
            gb = b % NBUF

            gather_wait(gb)

            if b < NBUF:
                @pl.when(grp > 0)
                def _():
                    store_wait(u - NBUF, gb)
            else:
                store_wait(u - NBUF, gb)

            transpose_scale(gb)
            store_start(u, gb)

            @pl.when(grp < N_GROUPS - 1)
            def _():
                icopy_start(u + INBUF, b)

            nb = (b + NBUF) % INBUF
            if b < NBUF:
                icopy_wait(nb)
                gather_start(nb, gb)
            else:
                @pl.when(grp < N_GROUPS - 1)
                def _():
                    icopy_wait(nb)
                    gather_start(nb, gb)

        return 0

    lax.fori_loop(0, N_GROUPS, group_body, 0)
    for b in range(NBUF):
        store_wait(N_UNITS - NBUF + b, b)


@jax.jit
def kernel(x, table):
    idx4 = x.T.reshape(25, 8, 32, CHUNK).transpose(0, 2, 1, 3)

    mesh = plsc.VectorSubcoreMesh(core_axis_name="c", subcore_axis_name="s")

    out5 = pl.kernel(
        _sc_kernel,
        mesh=mesh,
        compiler_params=pltpu.CompilerParams(use_tc_tiling_on_sc=False,
                                             needs_layout_passes=False),
        out_type=jax.ShapeDtypeStruct((200, 8, S_CHUNKS, 8, CHUNK),
                                      jnp.float32),
        scratch_types=(
            [pltpu.VMEM((CHUNK,), jnp.int32) for _ in range(INBUF)]
            + [pltpu.VMEM((CHUNK, D), jnp.float32) for _ in range(NBUF)]
            + [pltpu.VMEM((D, CHUNK), jnp.float32) for _ in range(NBUF)]
            + [pltpu.SemaphoreType.DMA for _ in range(INBUF + 2 * NBUF)]
        ),
    )(idx4, table)

    out = out5.transpose(0, 1, 3, 2, 4).reshape(200, D, 4096)
    return out.transpose(2, 0, 1)
